# baseline (speedup 1.0000x reference)
import functools

import jax
import jax.numpy as jnp
from jax import lax
from jax.experimental import pallas as pl
from jax.experimental.pallas import tpu as pltpu
from jax.experimental.pallas import tpu_sc as plsc

N = 10000
E = 320000
H = 128
G = 16
HP = 128

NC, NS = 2, 16
NW = NC * NS
NB = 4
CH = 200
NP = 10112
RPT = NP // NS
CHS = 80


def _sc_gather_body(es, sl, x_hbm, row_hbm, out_hbm, *scratch):
    c = lax.axis_index("c")
    s = lax.axis_index("s")
    wid = s * NC + c
    epw = es // NW
    base = wid * epw
    nch = epw // CH
    idx = scratch[:NB]
    rows = scratch[NB:2 * NB]
    gsem = scratch[2 * NB:3 * NB]
    wsem = scratch[3 * NB:]

    def off(i):
        return pl.multiple_of(base + i * CH, 8)

    def start(i, k):
        src = pl.multiple_of(sl + off(i), 8)
        pltpu.sync_copy(row_hbm.at[pl.ds(src, CH)], idx[k])
        pltpu.async_copy(x_hbm.at[idx[k]], rows[k], gsem[k])

    def finish(i, k):
        pltpu.make_async_copy(x_hbm.at[idx[k]], rows[k], gsem[k]).wait()
        pltpu.async_copy(rows[k], out_hbm.at[pl.ds(off(i), CH)], wsem[k])

    for i in range(min(NB, nch)):
        start(i, i)
    for i in range(nch):
        finish(i, i % NB)
        j = i - 2
        if j >= 0 and j + NB < nch:
            k = j % NB
            pltpu.make_async_copy(
                rows[k], out_hbm.at[pl.ds(off(j), CH)], wsem[k]).wait()
            start(j + NB, k)
    for i in range(max(0, nch - NB - 2), nch):
        k = i % NB
        if i + NB >= nch:
            pltpu.make_async_copy(
                rows[k], out_hbm.at[pl.ds(off(i), CH)], wsem[k]).wait()


@functools.cache
def _gather_call(es, sl):
    return pl.kernel(
        functools.partial(_sc_gather_body, es, sl),
        out_type=jax.ShapeDtypeStruct((es, H), jnp.float32),
        mesh=plsc.VectorSubcoreMesh(core_axis_name="c", subcore_axis_name="s"),
        scratch_types=(
            [pltpu.VMEM((CH,), jnp.int32)] * NB
            + [pltpu.VMEM((CH, H), jnp.float32)] * NB
            + [pltpu.SemaphoreType.DMA] * (2 * NB)
        ),
    )


def _sc_scatter_body(es, sl, msg_hbm, col_hbm, zero_hbm, out_hbm, *scratch):
    c = lax.axis_index("c")
    s = lax.axis_index("s")
    wid = s * NC + c
    epw = es // NW
    base = wid * epw
    nch = epw // CHS
    rbase = pl.multiple_of(s * RPT, 8)
    idx = scratch[:NB]
    rows = scratch[NB:2 * NB]
    msem = scratch[2 * NB:3 * NB]
    asem = scratch[3 * NB:4 * NB]
    acc = scratch[4 * NB]

    pltpu.sync_copy(zero_hbm.at[pl.ds(rbase, RPT)], acc.at[pl.ds(rbase, RPT)])
    plsc.subcore_barrier()

    def off(i):
        return pl.multiple_of(base + i * CHS, 8)

    def start(i, k):
        src = pl.multiple_of(E + sl + off(i), 8)
        pltpu.sync_copy(col_hbm.at[pl.ds(src, CHS)], idx[k])
        pltpu.async_copy(msg_hbm.at[pl.ds(off(i), CHS)], rows[k], msem[k])

    def finish(i, k):
        pltpu.make_async_copy(
            msg_hbm.at[pl.ds(off(i), CHS)], rows[k], msem[k]).wait()
        pltpu.async_copy(rows[k], acc.at[idx[k]], asem[k], add=True)

    for i in range(min(NB, nch)):
        start(i, i)
    for i in range(nch):
        finish(i, i % NB)
        j = i - 2
        if j >= 0 and j + NB < nch:
            k = j % NB
            pltpu.make_async_copy(rows[k], acc.at[idx[k]], asem[k]).wait()
            start(j + NB, k)
    for i in range(max(0, nch - NB - 2), nch):
        k = i % NB
        if i + NB >= nch:
            pltpu.make_async_copy(rows[k], acc.at[idx[k]], asem[k]).wait()

    plsc.subcore_barrier()
    pltpu.sync_copy(acc.at[pl.ds(rbase, RPT)], out_hbm.at[c, pl.ds(rbase, RPT)])


@functools.cache
def _scatter_call(es, sl):
    return pl.kernel(
        functools.partial(_sc_scatter_body, es, sl),
        out_type=jax.ShapeDtypeStruct((NC, NP, H), jnp.float32),
        mesh=plsc.VectorSubcoreMesh(core_axis_name="c", subcore_axis_name="s"),
        scratch_types=(
            [pltpu.VMEM((CHS,), jnp.int32)] * NB
            + [pltpu.VMEM((CHS, H), jnp.float32)] * NB
            + [pltpu.SemaphoreType.DMA] * (2 * NB)
            + [pltpu.VMEM_SHARED((NP, H), jnp.float32)]
        ),
    )


def _layer_norm_in(h, g, b):
    mu = jnp.mean(h, axis=-1, keepdims=True)
    var = jnp.mean((h - mu) ** 2, axis=-1, keepdims=True)
    return (h - mu) * lax.rsqrt(var + 1e-5) * g + b


def _edge_mlp_body(xg, ea, w0a, w0b, b0, w1, b1, w2, b2, w3, b3, g, be, out):
    f32 = jnp.float32
    bf = jnp.bfloat16
    h = jnp.dot(xg[...].astype(bf), w0a[...], preferred_element_type=f32)
    h = h + jnp.dot(ea[...].astype(bf), w0b[...], preferred_element_type=f32)
    h = jnp.maximum(h + b0[...], 0.0)
    h = jnp.maximum(
        jnp.dot(h.astype(bf), w1[...], preferred_element_type=f32) + b1[...], 0.0)
    h = jnp.maximum(
        jnp.dot(h.astype(bf), w2[...], preferred_element_type=f32) + b2[...], 0.0)
    h = jnp.dot(h.astype(bf), w3[...], preferred_element_type=f32) + b3[...]
    out[...] = _layer_norm_in(h, g[...], be[...])


BE = 2560
SLICES = (38400, 64000, 64000, 64000, 64000, 25600)
NSL = len(SLICES)


def _edge_mlp_call(sl, xg, ea, w0a, w0b, b0, w1, b1, w2, b2, w3, b3, g, be):
    es = xg.shape[0]
    blk0 = sl // BE
    full = lambda shape: pl.BlockSpec(shape, lambda i: (0,) * len(shape))
    return pl.pallas_call(
        _edge_mlp_body,
        grid=(es // BE,),
        in_specs=[
            pl.BlockSpec((BE, H), lambda i: (i, 0)),
            pl.BlockSpec((BE, H), lambda i: (blk0 + i, 0)),
            full((H, HP)), full((H, HP)), full((1, HP)),
            full((HP, HP)), full((1, HP)),
            full((HP, HP)), full((1, HP)),
            full((HP, H)), full((1, H)),
            full((1, H)), full((1, H)),
        ],
        out_specs=pl.BlockSpec((BE, H), lambda i: (i, 0)),
        out_shape=jax.ShapeDtypeStruct((es, H), jnp.float32),
    )(xg, ea, w0a, w0b, b0, w1, b1, w2, b2, w3, b3, g, be)


def _node_mlp_body(x, p0, p1, p2, p3, p4, p5, boh, u, w0a, w0b, w0c, b0,
                   w1, b1, w2, b2, w3, b3, g, be, out):
    f32 = jnp.float32
    agg = (p0[0] + p0[1]) + (p1[0] + p1[1]) + (p2[0] + p2[1]) \
        + (p3[0] + p3[1]) + (p4[0] + p4[1]) + (p5[0] + p5[1])
    uw = jnp.dot(u[...], w0c[...], preferred_element_type=f32)
    h = jnp.dot(x[...], w0a[...], preferred_element_type=f32)
    h = h + jnp.dot(agg, w0b[...], preferred_element_type=f32)
    h = h + jnp.dot(boh[...], uw, preferred_element_type=f32)
    h = jnp.maximum(h + b0[...], 0.0)
    h = jnp.maximum(jnp.dot(h, w1[...], preferred_element_type=f32) + b1[...], 0.0)
    h = jnp.maximum(jnp.dot(h, w2[...], preferred_element_type=f32) + b2[...], 0.0)
    h = jnp.dot(h, w3[...], preferred_element_type=f32) + b3[...]
    out[...] = x[...] + _layer_norm_in(h, g[...], be[...])


BN = 2000


def _node_mlp_call(x, ps, boh, u, w0a, w0b, w0c, b0, w1, b1, w2, b2, w3, b3,
                   g, be):
    full = lambda shape: pl.BlockSpec(shape, lambda i: (0,) * len(shape))
    pspec = pl.BlockSpec((NC, BN, H), lambda i: (0, i, 0))
    return pl.pallas_call(
        _node_mlp_body,
        grid=(N // BN,),
        in_specs=[
            pl.BlockSpec((BN, H), lambda i: (i, 0)),
            pspec, pspec, pspec, pspec, pspec, pspec,
            pl.BlockSpec((BN, G), lambda i: (i, 0)),
            full((G, H)),
            full((H, HP)), full((H, HP)), full((H, HP)), full((1, HP)),
            full((HP, HP)), full((1, HP)),
            full((HP, HP)), full((1, HP)),
            full((HP, H)), full((1, H)),
            full((1, H)), full((1, H)),
        ],
        out_specs=pl.BlockSpec((BN, H), lambda i: (i, 0)),
        out_shape=jax.ShapeDtypeStruct((N, H), jnp.float32),
    )(x, *ps, boh, u, w0a, w0b, w0c, b0, w1, b1, w2, b2, w3, b3, g, be)


def _pad_mat(w, rows, cols):
    return jnp.pad(w, ((0, rows - w.shape[0]), (0, cols - w.shape[1])))


def _pad_vec(b, cols):
    return jnp.pad(b, (0, cols - b.shape[0])).reshape(1, cols)


@jax.jit
def kernel(x, edge_index, edge_attr, u, batch,
           m1_W0, m1_b0, m1_W1, m1_b1, m1_W2, m1_b2, m1_W3, m1_b3,
           m1_ln_g, m1_ln_b,
           m2_W0, m2_b0, m2_W1, m2_b1, m2_W2, m2_b2, m2_W3, m2_b3,
           m2_ln_g, m2_ln_b):
    bf = jnp.bfloat16
    e_w0a = _pad_mat(m1_W0[:H], H, HP).astype(bf)
    e_w0b = _pad_mat(m1_W0[H:], H, HP).astype(bf)
    e_b0 = _pad_vec(m1_b0, HP)
    e_w1 = _pad_mat(m1_W1, HP, HP).astype(bf)
    e_b1 = _pad_vec(m1_b1, HP)
    e_w2 = _pad_mat(m1_W2, HP, HP).astype(bf)
    e_b2 = _pad_vec(m1_b2, HP)
    e_w3 = _pad_mat(m1_W3, HP, H).astype(bf)
    e_b3 = m1_b3.reshape(1, H)
    e_g = m1_ln_g.reshape(1, H)
    e_be = m1_ln_b.reshape(1, H)

    n_w0a = _pad_mat(m2_W0[:H], H, HP)
    n_w0b = _pad_mat(m2_W0[H:2 * H], H, HP)
    n_w0c = _pad_mat(m2_W0[2 * H:], H, HP)
    n_b0 = _pad_vec(m2_b0, HP)
    n_w1 = _pad_mat(m2_W1, HP, HP)
    n_b1 = _pad_vec(m2_b1, HP)
    n_w2 = _pad_mat(m2_W2, HP, HP)
    n_b2 = _pad_vec(m2_b2, HP)
    n_w3 = _pad_mat(m2_W3, HP, H)
    n_b3 = m2_b3.reshape(1, H)
    n_g = m2_ln_g.reshape(1, H)
    n_be = m2_ln_b.reshape(1, H)

    batch_oh = (batch[:, None] == jnp.arange(G, dtype=batch.dtype)[None, :])
    batch_oh = batch_oh.astype(jnp.float32)
    zeros = jnp.zeros((NP, H), jnp.float32)
    ei_flat = edge_index.reshape(2 * E)

    partials = []
    off0 = 0
    for es in SLICES:
        gathered = _gather_call(es, off0)(x, ei_flat)
        msg = _edge_mlp_call(off0, gathered, edge_attr, e_w0a, e_w0b, e_b0,
                             e_w1, e_b1, e_w2, e_b2, e_w3, e_b3, e_g, e_be)
        partials.append(_scatter_call(es, off0)(msg, ei_flat, zeros))
        off0 += es
    return _node_mlp_call(x, partials, batch_oh, u, n_w0a, n_w0b, n_w0c,
                          n_b0, n_w1, n_b1, n_w2, n_b2, n_w3, n_b3,
                          n_g, n_be)


# speedup vs baseline: 1.0533x; 1.0533x over previous
import functools

import jax
import jax.numpy as jnp
from jax import lax
from jax.experimental import pallas as pl
from jax.experimental.pallas import tpu as pltpu
from jax.experimental.pallas import tpu_sc as plsc

N = 10000
E = 320000
H = 128
G = 16
HP = 128

NC, NS = 2, 16
NW = NC * NS
NB = 4
CH = 200
NP = 10112
RPT = NP // NS
CHS = 80


def _sc_gather_body(es, sl, x_hbm, row_hbm, out_hbm, *scratch):
    c = lax.axis_index("c")
    s = lax.axis_index("s")
    wid = s * NC + c
    epw = es // NW
    base = wid * epw
    nch = epw // CH
    idx = scratch[:NB]
    rows = scratch[NB:2 * NB]
    gsem = scratch[2 * NB:3 * NB]
    wsem = scratch[3 * NB:]

    def off(i):
        return pl.multiple_of(base + i * CH, 8)

    def start(i, k):
        src = pl.multiple_of(sl + off(i), 8)
        pltpu.sync_copy(row_hbm.at[pl.ds(src, CH)], idx[k])
        pltpu.async_copy(x_hbm.at[idx[k]], rows[k], gsem[k])

    def finish(i, k):
        pltpu.make_async_copy(x_hbm.at[idx[k]], rows[k], gsem[k]).wait()
        pltpu.async_copy(rows[k], out_hbm.at[pl.ds(off(i), CH)], wsem[k])

    for i in range(min(NB, nch)):
        start(i, i)
    for i in range(nch):
        finish(i, i % NB)
        j = i - 2
        if j >= 0 and j + NB < nch:
            k = j % NB
            pltpu.make_async_copy(
                rows[k], out_hbm.at[pl.ds(off(j), CH)], wsem[k]).wait()
            start(j + NB, k)
    for i in range(max(0, nch - NB - 2), nch):
        k = i % NB
        if i + NB >= nch:
            pltpu.make_async_copy(
                rows[k], out_hbm.at[pl.ds(off(i), CH)], wsem[k]).wait()


@functools.cache
def _gather_call(es, sl):
    return pl.kernel(
        functools.partial(_sc_gather_body, es, sl),
        out_type=jax.ShapeDtypeStruct((es, H), jnp.float32),
        mesh=plsc.VectorSubcoreMesh(core_axis_name="c", subcore_axis_name="s"),
        scratch_types=(
            [pltpu.VMEM((CH,), jnp.int32)] * NB
            + [pltpu.VMEM((CH, H), jnp.float32)] * NB
            + [pltpu.SemaphoreType.DMA] * (2 * NB)
        ),
    )


def _sc_scatter_body(es, sl, first, msg_hbm, col_hbm, init_hbm, out_hbm,
                     *scratch):
    c = lax.axis_index("c")
    s = lax.axis_index("s")
    wid = s * NC + c
    epw = es // NW
    base = wid * epw
    nch = epw // CHS
    rbase = pl.multiple_of(s * RPT, 8)
    idx = scratch[:NB]
    rows = scratch[NB:2 * NB]
    msem = scratch[2 * NB:3 * NB]
    asem = scratch[3 * NB:4 * NB]
    acc = scratch[4 * NB]

    if first:
        pltpu.sync_copy(init_hbm.at[pl.ds(rbase, RPT)],
                        acc.at[pl.ds(rbase, RPT)])
    else:
        pltpu.sync_copy(init_hbm.at[c, pl.ds(rbase, RPT)],
                        acc.at[pl.ds(rbase, RPT)])
    plsc.subcore_barrier()

    def off(i):
        return pl.multiple_of(base + i * CHS, 8)

    def start(i, k):
        src = pl.multiple_of(E + sl + off(i), 8)
        pltpu.sync_copy(col_hbm.at[pl.ds(src, CHS)], idx[k])
        pltpu.async_copy(msg_hbm.at[pl.ds(off(i), CHS)], rows[k], msem[k])

    def finish(i, k):
        pltpu.make_async_copy(
            msg_hbm.at[pl.ds(off(i), CHS)], rows[k], msem[k]).wait()
        pltpu.async_copy(rows[k], acc.at[idx[k]], asem[k], add=True)

    for i in range(min(NB, nch)):
        start(i, i)
    for i in range(nch):
        finish(i, i % NB)
        j = i - 2
        if j >= 0 and j + NB < nch:
            k = j % NB
            pltpu.make_async_copy(rows[k], acc.at[idx[k]], asem[k]).wait()
            start(j + NB, k)
    for i in range(max(0, nch - NB - 2), nch):
        k = i % NB
        if i + NB >= nch:
            pltpu.make_async_copy(rows[k], acc.at[idx[k]], asem[k]).wait()

    plsc.subcore_barrier()
    pltpu.sync_copy(acc.at[pl.ds(rbase, RPT)], out_hbm.at[c, pl.ds(rbase, RPT)])


@functools.cache
def _scatter_call(es, sl, first):
    return pl.kernel(
        functools.partial(_sc_scatter_body, es, sl, first),
        out_type=jax.ShapeDtypeStruct((NC, NP, H), jnp.float32),
        mesh=plsc.VectorSubcoreMesh(core_axis_name="c", subcore_axis_name="s"),
        scratch_types=(
            [pltpu.VMEM((CHS,), jnp.int32)] * NB
            + [pltpu.VMEM((CHS, H), jnp.float32)] * NB
            + [pltpu.SemaphoreType.DMA] * (2 * NB)
            + [pltpu.VMEM_SHARED((NP, H), jnp.float32)]
        ),
    )


def _layer_norm_in(h, g, b):
    mu = jnp.mean(h, axis=-1, keepdims=True)
    var = jnp.mean((h - mu) ** 2, axis=-1, keepdims=True)
    return (h - mu) * lax.rsqrt(var + 1e-5) * g + b


def _edge_mlp_body(xg, ea, w0a, w0b, b0, w1, b1, w2, b2, w3, b3, g, be, out):
    f32 = jnp.float32
    bf = jnp.bfloat16
    h = jnp.dot(xg[...].astype(bf), w0a[...], preferred_element_type=f32)
    h = h + jnp.dot(ea[...].astype(bf), w0b[...], preferred_element_type=f32)
    h = jnp.maximum(h + b0[...], 0.0)
    h = jnp.maximum(
        jnp.dot(h.astype(bf), w1[...], preferred_element_type=f32) + b1[...], 0.0)
    h = jnp.maximum(
        jnp.dot(h.astype(bf), w2[...], preferred_element_type=f32) + b2[...], 0.0)
    h = jnp.dot(h.astype(bf), w3[...], preferred_element_type=f32) + b3[...]
    out[...] = _layer_norm_in(h, g[...], be[...])


BE = 2560
SLICES = (64000, 64000, 64000, 64000, 64000)
NSL = len(SLICES)


def _edge_mlp_call(sl, xg, ea, w0a, w0b, b0, w1, b1, w2, b2, w3, b3, g, be):
    es = xg.shape[0]
    blk0 = sl // BE
    full = lambda shape: pl.BlockSpec(shape, lambda i: (0,) * len(shape))
    return pl.pallas_call(
        _edge_mlp_body,
        grid=(es // BE,),
        in_specs=[
            pl.BlockSpec((BE, H), lambda i: (i, 0)),
            pl.BlockSpec((BE, H), lambda i: (blk0 + i, 0)),
            full((H, HP)), full((H, HP)), full((1, HP)),
            full((HP, HP)), full((1, HP)),
            full((HP, HP)), full((1, HP)),
            full((HP, H)), full((1, H)),
            full((1, H)), full((1, H)),
        ],
        out_specs=pl.BlockSpec((BE, H), lambda i: (i, 0)),
        out_shape=jax.ShapeDtypeStruct((es, H), jnp.float32),
    )(xg, ea, w0a, w0b, b0, w1, b1, w2, b2, w3, b3, g, be)


def _node_mlp_body(x, p, boh, u, w0a, w0b, w0c, b0,
                   w1, b1, w2, b2, w3, b3, g, be, out):
    f32 = jnp.float32
    agg = p[0] + p[1]
    uw = jnp.dot(u[...], w0c[...], preferred_element_type=f32)
    h = jnp.dot(x[...], w0a[...], preferred_element_type=f32)
    h = h + jnp.dot(agg, w0b[...], preferred_element_type=f32)
    h = h + jnp.dot(boh[...], uw, preferred_element_type=f32)
    h = jnp.maximum(h + b0[...], 0.0)
    h = jnp.maximum(jnp.dot(h, w1[...], preferred_element_type=f32) + b1[...], 0.0)
    h = jnp.maximum(jnp.dot(h, w2[...], preferred_element_type=f32) + b2[...], 0.0)
    h = jnp.dot(h, w3[...], preferred_element_type=f32) + b3[...]
    out[...] = x[...] + _layer_norm_in(h, g[...], be[...])


BN = 2000


def _node_mlp_call(x, ps, boh, u, w0a, w0b, w0c, b0, w1, b1, w2, b2, w3, b3,
                   g, be):
    full = lambda shape: pl.BlockSpec(shape, lambda i: (0,) * len(shape))
    pspec = pl.BlockSpec((NC, BN, H), lambda i: (0, i, 0))
    return pl.pallas_call(
        _node_mlp_body,
        grid=(N // BN,),
        in_specs=[
            pl.BlockSpec((BN, H), lambda i: (i, 0)),
            pspec,
            pl.BlockSpec((BN, G), lambda i: (i, 0)),
            full((G, H)),
            full((H, HP)), full((H, HP)), full((H, HP)), full((1, HP)),
            full((HP, HP)), full((1, HP)),
            full((HP, HP)), full((1, HP)),
            full((HP, H)), full((1, H)),
            full((1, H)), full((1, H)),
        ],
        out_specs=pl.BlockSpec((BN, H), lambda i: (i, 0)),
        out_shape=jax.ShapeDtypeStruct((N, H), jnp.float32),
    )(x, ps, boh, u, w0a, w0b, w0c, b0, w1, b1, w2, b2, w3, b3, g, be)


def _pad_mat(w, rows, cols):
    return jnp.pad(w, ((0, rows - w.shape[0]), (0, cols - w.shape[1])))


def _pad_vec(b, cols):
    return jnp.pad(b, (0, cols - b.shape[0])).reshape(1, cols)


@jax.jit
def kernel(x, edge_index, edge_attr, u, batch,
           m1_W0, m1_b0, m1_W1, m1_b1, m1_W2, m1_b2, m1_W3, m1_b3,
           m1_ln_g, m1_ln_b,
           m2_W0, m2_b0, m2_W1, m2_b1, m2_W2, m2_b2, m2_W3, m2_b3,
           m2_ln_g, m2_ln_b):
    bf = jnp.bfloat16
    e_w0a = _pad_mat(m1_W0[:H], H, HP).astype(bf)
    e_w0b = _pad_mat(m1_W0[H:], H, HP).astype(bf)
    e_b0 = _pad_vec(m1_b0, HP)
    e_w1 = _pad_mat(m1_W1, HP, HP).astype(bf)
    e_b1 = _pad_vec(m1_b1, HP)
    e_w2 = _pad_mat(m1_W2, HP, HP).astype(bf)
    e_b2 = _pad_vec(m1_b2, HP)
    e_w3 = _pad_mat(m1_W3, HP, H).astype(bf)
    e_b3 = m1_b3.reshape(1, H)
    e_g = m1_ln_g.reshape(1, H)
    e_be = m1_ln_b.reshape(1, H)

    n_w0a = _pad_mat(m2_W0[:H], H, HP)
    n_w0b = _pad_mat(m2_W0[H:2 * H], H, HP)
    n_w0c = _pad_mat(m2_W0[2 * H:], H, HP)
    n_b0 = _pad_vec(m2_b0, HP)
    n_w1 = _pad_mat(m2_W1, HP, HP)
    n_b1 = _pad_vec(m2_b1, HP)
    n_w2 = _pad_mat(m2_W2, HP, HP)
    n_b2 = _pad_vec(m2_b2, HP)
    n_w3 = _pad_mat(m2_W3, HP, H)
    n_b3 = m2_b3.reshape(1, H)
    n_g = m2_ln_g.reshape(1, H)
    n_be = m2_ln_b.reshape(1, H)

    batch_oh = (batch[:, None] == jnp.arange(G, dtype=batch.dtype)[None, :])
    batch_oh = batch_oh.astype(jnp.float32)
    zeros = jnp.zeros((NP, H), jnp.float32)
    ei_flat = edge_index.reshape(2 * E)

    partial = zeros
    off0 = 0
    for i, es in enumerate(SLICES):
        gathered = _gather_call(es, off0)(x, ei_flat)
        msg = _edge_mlp_call(off0, gathered, edge_attr, e_w0a, e_w0b, e_b0,
                             e_w1, e_b1, e_w2, e_b2, e_w3, e_b3, e_g, e_be)
        partial = _scatter_call(es, off0, i == 0)(msg, ei_flat, partial)
        off0 += es
    return _node_mlp_call(x, partial, batch_oh, u, n_w0a, n_w0b, n_w0c,
                          n_b0, n_w1, n_b1, n_w2, n_b2, n_w3, n_b3,
                          n_g, n_be)
